# baseline (speedup 1.0000x reference)
import functools

import jax
import jax.numpy as jnp
from jax import lax
from jax.experimental import pallas as pl
from jax.experimental.pallas import tpu as pltpu
from jax.experimental.pallas import tpu_sc as plsc

VOCAB = 400000
EMB = 100
EMB_PAD = 104
HID = 128
NUM_CLASSES = 4
BATCH = 4096
SEQ = 200

NUM_WORKERS = 32
B_PER_W = BATCH // NUM_WORKERS


def _pool_kernel_body(x_hbm, table_hbm, out_hbm,
                      x_v, idx0, idx1, rows0, rows1, out_v, sem0, sem1):
    cid = lax.axis_index("c")
    sid = lax.axis_index("s")
    wid = sid * 2 + cid
    base = wid * B_PER_W

    pltpu.sync_copy(x_hbm.at[pl.ds(base, B_PER_W)], x_v)

    inv_seq = jnp.float32(1.0 / SEQ)

    def fire(idx_row, rows_v, sem, b):
        for c in range(12):
            idx_row[pl.ds(c * 16, 16)] = x_v[b, pl.ds(c * 16, 16)]
        idx_row[pl.ds(184, 16)] = x_v[b, pl.ds(184, 16)]
        pltpu.async_copy(table_hbm.at[idx_row.at[pl.ds(0, 128)]],
                         rows_v.at[pl.ds(0, 128)], sem)
        pltpu.async_copy(table_hbm.at[idx_row.at[pl.ds(128, 72)]],
                         rows_v.at[pl.ds(128, 72)], sem)

    def drain(idx_row, rows_v, sem):
        pltpu.make_async_copy(table_hbm.at[idx_row.at[pl.ds(0, 128)]],
                              rows_v.at[pl.ds(0, 128)], sem).wait()
        pltpu.make_async_copy(table_hbm.at[idx_row.at[pl.ds(128, 72)]],
                              rows_v.at[pl.ds(128, 72)], sem).wait()

    def accum_store(rows_v, b):
        def seq_body(t, accs):
            s = t * 4
            for u in range(4):
                new = [accs[c] + rows_v[s + u, pl.ds(c * 16, 16)]
                       for c in range(6)]
                new.append(accs[6] + rows_v[s + u, pl.ds(88, 16)])
                accs = tuple(new)
            return accs

        zero = jnp.zeros((16,), jnp.float32)
        accs = lax.fori_loop(0, SEQ // 4, seq_body, (zero,) * 7)
        for c in range(6):
            out_v[b, pl.ds(c * 16, 16)] = accs[c] * inv_seq
        out_v[b, pl.ds(88, 16)] = accs[6] * inv_seq

    fire(idx0, rows0, sem0, 0)

    def body(k, carry):
        b = 2 * k
        fire(idx1, rows1, sem1, b + 1)
        drain(idx0, rows0, sem0)
        accum_store(rows0, b)

        @pl.when(b + 2 < B_PER_W)
        def _():
            fire(idx0, rows0, sem0, b + 2)

        drain(idx1, rows1, sem1)
        accum_store(rows1, b + 1)
        return carry

    lax.fori_loop(0, B_PER_W // 2, body, 0)

    pltpu.sync_copy(out_v, out_hbm.at[pl.ds(base, B_PER_W)])


@jax.jit
def _pool(x, table):
    mesh = plsc.VectorSubcoreMesh(core_axis_name="c", subcore_axis_name="s")
    fn = functools.partial(
        pl.kernel,
        mesh=mesh,
        out_type=jax.ShapeDtypeStruct((BATCH, EMB_PAD), jnp.float32),
        scratch_types=[
            pltpu.VMEM((B_PER_W, SEQ), jnp.int32),
            pltpu.VMEM((256,), jnp.int32),
            pltpu.VMEM((256,), jnp.int32),
            pltpu.VMEM((SEQ, EMB_PAD), jnp.float32),
            pltpu.VMEM((SEQ, EMB_PAD), jnp.float32),
            pltpu.VMEM((B_PER_W, EMB_PAD), jnp.float32),
            pltpu.SemaphoreType.DMA,
            pltpu.SemaphoreType.DMA,
        ],
        compiler_params=pltpu.CompilerParams(use_tc_tiling_on_sc=False),
    )(_pool_kernel_body)
    return fn(x, table)


def _mlp_body(p_ref, w1_ref, b1_ref, w2_ref, b2_ref, o_ref):
    p = p_ref[...][:, :EMB]
    h = jnp.dot(p, w1_ref[...], preferred_element_type=jnp.float32)
    h = jnp.maximum(h + b1_ref[...], 0.0)
    o = jnp.dot(h, w2_ref[...], preferred_element_type=jnp.float32)
    o_ref[...] = o + b2_ref[...]


@jax.jit
def _mlp(pooled, W1, b1, W2, b2):
    return pl.pallas_call(
        _mlp_body,
        out_shape=jax.ShapeDtypeStruct((BATCH, NUM_CLASSES), jnp.float32),
    )(pooled, W1, b1.reshape(1, HID), W2, b2.reshape(1, NUM_CLASSES))


def kernel(x, table, W1, b1, W2, b2):
    table_pad = jnp.pad(table, ((0, 0), (0, EMB_PAD - EMB)))
    pooled = _pool(x, table_pad)
    return _mlp(pooled, W1, b1, W2, b2)


# speedup vs baseline: 1.6428x; 1.6428x over previous
import functools

import jax
import jax.numpy as jnp
from jax import lax
from jax.experimental import pallas as pl
from jax.experimental.pallas import tpu as pltpu
from jax.experimental.pallas import tpu_sc as plsc

VOCAB = 400000
EMB = 100
EMB_PAD = 128
HID = 128
NUM_CLASSES = 4
BATCH = 4096
SEQ = 200

NUM_WORKERS = 32
B_PER_W = BATCH // NUM_WORKERS
SEQ_PAD = 208


def _pool_kernel_body(x_hbm, table_hbm, out_hbm,
                      x_v, idx0, idx1, idx2, rows0, rows1, rows2, out_v,
                      sem0, sem1, sem2):
    cid = lax.axis_index("c")
    sid = lax.axis_index("s")
    wid = sid * 2 + cid
    base = wid * B_PER_W
    idxs = (idx0, idx1, idx2)
    rows = (rows0, rows1, rows2)
    sems = (sem0, sem1, sem2)

    pltpu.sync_copy(x_hbm.at[pl.ds(base, B_PER_W)], x_v)

    inv_seq = jnp.float32(1.0 / SEQ)

    def fire(idx_row, rows_v, sem, b):
        for c in range(12):
            idx_row[0, pl.ds(c * 16, 16)] = x_v[b, pl.ds(c * 16, 16)]
        idx_row[0, pl.ds(184, 16)] = x_v[b, pl.ds(184, 16)]
        pltpu.async_copy(table_hbm.at[idx_row.at[0, pl.ds(0, 128)]],
                         rows_v.at[pl.ds(0, 128)], sem)
        pltpu.async_copy(table_hbm.at[idx_row.at[0, pl.ds(128, 72)]],
                         rows_v.at[pl.ds(128, 72)], sem)

    def drain(idx_row, rows_v, sem):
        pltpu.make_async_copy(table_hbm.at[idx_row.at[0, pl.ds(0, 128)]],
                              rows_v.at[pl.ds(0, 128)], sem).wait()
        pltpu.make_async_copy(table_hbm.at[idx_row.at[0, pl.ds(128, 72)]],
                              rows_v.at[pl.ds(128, 72)], sem).wait()

    def accum_store(rows_v, b):
        def seq_body(t, accs):
            s = t * 4
            for u in range(4):
                new = [accs[c] + rows_v[s + u, pl.ds(c * 16, 16)]
                       for c in range(6)]
                new.append(accs[6] + rows_v[s + u, pl.ds(84, 16)])
                accs = tuple(new)
            return accs

        @pl.when(b == B_PER_W // 2)
        def _():
            pltpu.sync_copy(out_v, out_hbm.at[pl.ds(base, B_PER_W // 2)])

        zero = jnp.zeros((16,), jnp.float32)
        accs = lax.fori_loop(0, SEQ // 4, seq_body, (zero,) * 7)
        slot = lax.rem(b, B_PER_W // 2)
        for c in range(6):
            out_v[slot, pl.ds(c * 16, 16)] = accs[c] * inv_seq
        out_v[slot, pl.ds(84, 16)] = accs[6] * inv_seq

    fire(idx0, rows0, sem0, 0)
    fire(idx1, rows1, sem1, 1)

    def body(j, carry):
        for t in range(3):
            b = 3 * j + t
            fire(idxs[(t + 2) % 3], rows[(t + 2) % 3], sems[(t + 2) % 3],
                 b + 2)
            drain(idxs[t], rows[t], sems[t])
            accum_store(rows[t], b)
        return carry

    lax.fori_loop(0, (B_PER_W - 2) // 3, body, 0)

    drain(idx0, rows0, sem0)
    accum_store(rows0, B_PER_W - 2)
    drain(idx1, rows1, sem1)
    accum_store(rows1, B_PER_W - 1)

    pltpu.sync_copy(out_v, out_hbm.at[pl.ds(base + B_PER_W // 2,
                                            B_PER_W // 2)])


@jax.jit
def _pool(x, table):
    mesh = plsc.VectorSubcoreMesh(core_axis_name="c", subcore_axis_name="s")
    fn = functools.partial(
        pl.kernel,
        mesh=mesh,
        out_type=jax.ShapeDtypeStruct((BATCH, EMB_PAD), jnp.float32),
        scratch_types=[
            pltpu.VMEM((B_PER_W, SEQ), jnp.int32),
            pltpu.VMEM((8, 256), jnp.int32),
            pltpu.VMEM((8, 256), jnp.int32),
            pltpu.VMEM((8, 256), jnp.int32),
            pltpu.VMEM((SEQ, EMB_PAD), jnp.float32),
            pltpu.VMEM((SEQ, EMB_PAD), jnp.float32),
            pltpu.VMEM((SEQ, EMB_PAD), jnp.float32),
            pltpu.VMEM((B_PER_W // 2, EMB_PAD), jnp.float32),
            pltpu.SemaphoreType.DMA,
            pltpu.SemaphoreType.DMA,
            pltpu.SemaphoreType.DMA,
        ],
    )(_pool_kernel_body)
    return fn(x, table)


def _mlp_body(p_ref, w1_ref, b1_ref, w2_ref, b2_ref, o_ref):
    p = p_ref[...][:, :EMB]
    h = jnp.dot(p, w1_ref[...], preferred_element_type=jnp.float32)
    h = jnp.maximum(h + b1_ref[...], 0.0)
    o = jnp.dot(h, w2_ref[...], preferred_element_type=jnp.float32)
    o_ref[...] = o + b2_ref[...]


@jax.jit
def _mlp(pooled, W1, b1, W2, b2):
    return pl.pallas_call(
        _mlp_body,
        out_shape=jax.ShapeDtypeStruct((BATCH, NUM_CLASSES), jnp.float32),
    )(pooled, W1, b1.reshape(1, HID), W2, b2.reshape(1, NUM_CLASSES))


def kernel(x, table, W1, b1, W2, b2):
    table_pad = jnp.pad(table, ((0, 0), (0, EMB_PAD - EMB)))
    pooled = _pool(x, table_pad)
    return _mlp(pooled, W1, b1, W2, b2)
